# baseline (speedup 1.0000x reference)
import functools
import math

import jax
import jax.numpy as jnp
from jax.experimental import pallas as pl
from jax.experimental.pallas import tpu as pltpu

E = 4
R = 16
ALPHA = 32.0
SCALING = ALPHA / R
BALANCE_COEFF = 0.01

S_BLK = 1024


def _fused_body(inv_seq, x_ref, a_ref, b_ref, rel_ref, reg_ref, w1x_ref,
                w1r_ref, w1g_ref, b1_ref, w2_ref, b2_ref, erep_ref,
                out_ref, bal_ref, xa_s, pooled_s, gscale_s, probs_s):
    b = pl.program_id(0)
    p = pl.program_id(1)
    s = pl.program_id(2)
    nb = pl.num_programs(0)
    s_blk = x_ref.shape[1]

    @pl.when(p == 0)
    def _pass1():
        xb = x_ref[0]
        xa_s[pl.ds(s * s_blk, s_blk)] = jax.lax.dot_general(
            xb, a_ref[...], (((1,), (1,)), ((), ())),
            preferred_element_type=jnp.float32)
        psum = jnp.sum(xb, axis=0, keepdims=True)

        @pl.when(s == 0)
        def _init():
            pooled_s[...] = psum

        @pl.when(s != 0)
        def _acc():
            pooled_s[...] += psum

    @pl.when((p == 1) & (s == 0))
    def _router():
        pooled = pooled_s[...] * inv_seq
        h = jax.lax.dot_general(pooled, w1x_ref[...], (((1,), (1,)), ((), ())),
                                preferred_element_type=jnp.float32)
        h += jax.lax.dot_general(rel_ref[pl.ds(b, 1)], w1r_ref[...],
                                 (((1,), (1,)), ((), ())),
                                 preferred_element_type=jnp.float32)
        h += jax.lax.dot_general(reg_ref[pl.ds(b, 1)], w1g_ref[...],
                                 (((1,), (1,)), ((), ())),
                                 preferred_element_type=jnp.float32)
        h += b1_ref[...]
        h = 0.5 * h * (1.0 + jax.lax.erf(h * (1.0 / math.sqrt(2.0))))
        logits = jax.lax.dot_general(h, w2_ref[...], (((1,), (1,)), ((), ())),
                                     preferred_element_type=jnp.float32)
        logits += b2_ref[...]
        m = jnp.max(logits, axis=-1, keepdims=True)
        pr = jnp.exp(logits - m)
        pr = pr / jnp.sum(pr, axis=-1, keepdims=True)
        probs_s[pl.ds(b, 1)] = pr
        gscale_s[...] = jax.lax.dot_general(
            pr, erep_ref[...], (((1,), (0,)), ((), ())),
            preferred_element_type=jnp.float32) * SCALING

        @pl.when(b == nb - 1)
        def _balance():
            avg = jnp.mean(probs_s[...], axis=0, keepdims=True)
            bal_ref[...] = BALANCE_COEFF * E * jnp.sum(avg * avg, axis=1,
                                                       keepdims=True)

    @pl.when(p == 1)
    def _pass2():
        xs = xa_s[pl.ds(s * s_blk, s_blk)] * gscale_s[...]
        out_ref[0] = jax.lax.dot_general(
            xs, b_ref[...], (((1,), (0,)), ((), ())),
            preferred_element_type=jnp.float32)


def kernel(x, reliability_vec, regime_vec, lora_A, lora_B, W1, b1, W2, b2):
    B, S, d_in = x.shape
    e, r, _ = lora_A.shape
    er = e * r
    d_out = lora_B.shape[1]
    hid = W1.shape[0]
    ns = S // S_BLK

    a_mat = lora_A.reshape(er, d_in)
    b_mat = lora_B.transpose(0, 2, 1).reshape(er, d_out)
    w1x = W1[:, :d_in]
    w1r = W1[:, d_in:d_in + reliability_vec.shape[1]]
    w1g = W1[:, d_in + reliability_vec.shape[1]:]
    b1_2d = b1.reshape(1, hid)
    b2_2d = b2.reshape(1, e)
    erep = jnp.repeat(jnp.eye(e, dtype=jnp.float32), r, axis=1)

    def x_map(bb, pp, ss):
        return (bb, jnp.where(pp == 0, ss, ns - 1), 0)

    def out_map(bb, pp, ss):
        return (bb, jnp.where(pp == 1, ss, 0), 0)

    small = [reliability_vec, regime_vec, w1x, w1r, w1g, b1_2d, W2, b2_2d,
             erep]
    out, bal = pl.pallas_call(
        functools.partial(_fused_body, 1.0 / S),
        grid=(B, 2, ns),
        in_specs=[pl.BlockSpec((1, S_BLK, d_in), x_map),
                  pl.BlockSpec((er, d_in), lambda bb, pp, ss: (0, 0)),
                  pl.BlockSpec((er, d_out), lambda bb, pp, ss: (0, 0))]
                 + [pl.BlockSpec(a.shape, lambda bb, pp, ss: (0, 0))
                    for a in small],
        out_specs=[
            pl.BlockSpec((1, S_BLK, d_out), out_map),
            pl.BlockSpec((1, 1), lambda bb, pp, ss: (0, 0)),
        ],
        out_shape=[
            jax.ShapeDtypeStruct((B, S, d_out), jnp.float32),
            jax.ShapeDtypeStruct((1, 1), jnp.float32),
        ],
        scratch_shapes=[
            pltpu.VMEM((S, er), jnp.float32),
            pltpu.VMEM((1, d_in), jnp.float32),
            pltpu.VMEM((1, er), jnp.float32),
            pltpu.VMEM((B, e), jnp.float32),
        ],
        compiler_params=pltpu.CompilerParams(
            dimension_semantics=("arbitrary", "arbitrary", "arbitrary")),
    )(x, a_mat, b_mat, *small)

    return (out, bal.reshape(()))


# speedup vs baseline: 1.1925x; 1.1925x over previous
import functools
import math

import jax
import jax.numpy as jnp
from jax.experimental import pallas as pl
from jax.experimental.pallas import tpu as pltpu

E = 4
R = 16
ALPHA = 32.0
SCALING = ALPHA / R
BALANCE_COEFF = 0.01

S_BLK = 1024


def _fused_body(inv_seq, x_ref, a_ref, b_ref, rel_ref, reg_ref, w1x_ref,
                w1r_ref, w1g_ref, b1_ref, w2_ref, b2_ref, erep_ref,
                out_ref, bal_ref, xa_s, pooled_s, gscale_s):
    p = pl.program_id(0)
    b = pl.program_id(1)
    s = pl.program_id(2)
    s_blk = x_ref.shape[1]

    @pl.when(p == 0)
    def _pass1():
        xb = x_ref[0]
        xa_s[b, pl.ds(s * s_blk, s_blk)] = jax.lax.dot_general(
            xb.astype(jnp.bfloat16), a_ref[...], (((1,), (1,)), ((), ())),
            preferred_element_type=jnp.float32)
        psum = jnp.sum(xb, axis=0, keepdims=True)

        @pl.when(s == 0)
        def _init():
            pooled_s[pl.ds(b, 1)] = psum

        @pl.when(s != 0)
        def _acc():
            pooled_s[pl.ds(b, 1)] += psum

    @pl.when((p == 1) & (b == 0) & (s == 0))
    def _router():
        pooled = pooled_s[...] * inv_seq
        h = jax.lax.dot_general(pooled, w1x_ref[...], (((1,), (1,)), ((), ())),
                                preferred_element_type=jnp.float32)
        h += jax.lax.dot_general(rel_ref[...], w1r_ref[...],
                                 (((1,), (1,)), ((), ())),
                                 preferred_element_type=jnp.float32)
        h += jax.lax.dot_general(reg_ref[...], w1g_ref[...],
                                 (((1,), (1,)), ((), ())),
                                 preferred_element_type=jnp.float32)
        h += b1_ref[...]
        h = 0.5 * h * (1.0 + jax.lax.erf(h * (1.0 / math.sqrt(2.0))))
        logits = jax.lax.dot_general(h, w2_ref[...], (((1,), (1,)), ((), ())),
                                     preferred_element_type=jnp.float32)
        logits += b2_ref[...]
        m = jnp.max(logits, axis=-1, keepdims=True)
        pr = jnp.exp(logits - m)
        pr = pr / jnp.sum(pr, axis=-1, keepdims=True)
        gscale_s[...] = jax.lax.dot_general(
            pr, erep_ref[...], (((1,), (0,)), ((), ())),
            preferred_element_type=jnp.float32) * SCALING
        avg = jnp.mean(pr, axis=0, keepdims=True)
        bal_ref[...] = BALANCE_COEFF * E * jnp.sum(avg * avg, axis=1,
                                                   keepdims=True)

    @pl.when(p == 1)
    def _pass2():
        xs = xa_s[b, pl.ds(s * s_blk, s_blk)] * gscale_s[pl.ds(b, 1)]
        out_ref[0] = jax.lax.dot_general(
            xs.astype(jnp.bfloat16), b_ref[...], (((1,), (0,)), ((), ())),
            preferred_element_type=jnp.float32)


def kernel(x, reliability_vec, regime_vec, lora_A, lora_B, W1, b1, W2, b2):
    B, S, d_in = x.shape
    e, r, _ = lora_A.shape
    er = e * r
    d_out = lora_B.shape[1]
    hid = W1.shape[0]
    ns = S // S_BLK

    a_mat = lora_A.reshape(er, d_in).astype(jnp.bfloat16)
    b_mat = lora_B.transpose(0, 2, 1).reshape(er, d_out).astype(jnp.bfloat16)
    w1x = W1[:, :d_in]
    w1r = W1[:, d_in:d_in + reliability_vec.shape[1]]
    w1g = W1[:, d_in + reliability_vec.shape[1]:]
    b1_2d = b1.reshape(1, hid)
    b2_2d = b2.reshape(1, e)
    erep = jnp.repeat(jnp.eye(e, dtype=jnp.float32), r, axis=1)

    def x_map(pp, bb, ss):
        return (jnp.where(pp == 0, bb, B - 1),
                jnp.where(pp == 0, ss, ns - 1), 0)

    def out_map(pp, bb, ss):
        return (jnp.where(pp == 1, bb, 0), jnp.where(pp == 1, ss, 0), 0)

    small = [reliability_vec, regime_vec, w1x, w1r, w1g, b1_2d, W2, b2_2d,
             erep]
    out, bal = pl.pallas_call(
        functools.partial(_fused_body, 1.0 / S),
        grid=(2, B, ns),
        in_specs=[pl.BlockSpec((1, S_BLK, d_in), x_map),
                  pl.BlockSpec((er, d_in), lambda pp, bb, ss: (0, 0)),
                  pl.BlockSpec((er, d_out), lambda pp, bb, ss: (0, 0))]
                 + [pl.BlockSpec(a.shape, lambda pp, bb, ss: (0, 0))
                    for a in small],
        out_specs=[
            pl.BlockSpec((1, S_BLK, d_out), out_map),
            pl.BlockSpec((1, 1), lambda pp, bb, ss: (0, 0)),
        ],
        out_shape=[
            jax.ShapeDtypeStruct((B, S, d_out), jnp.float32),
            jax.ShapeDtypeStruct((1, 1), jnp.float32),
        ],
        scratch_shapes=[
            pltpu.VMEM((B, S, er), jnp.float32),
            pltpu.VMEM((B, d_in), jnp.float32),
            pltpu.VMEM((B, er), jnp.float32),
        ],
        compiler_params=pltpu.CompilerParams(
            dimension_semantics=("arbitrary", "arbitrary", "arbitrary")),
    )(x, a_mat, b_mat, *small)

    return (out, bal.reshape(()))


# speedup vs baseline: 1.2749x; 1.0692x over previous
import functools
import math

import jax
import jax.numpy as jnp
from jax.experimental import pallas as pl
from jax.experimental.pallas import tpu as pltpu

E = 4
R = 16
ALPHA = 32.0
SCALING = ALPHA / R
BALANCE_COEFF = 0.01

S_BLK = 1024


def _fused_body(inv_seq, x_ref, a_ref, b_ref, rel_ref, reg_ref, w1x_ref,
                w1r_ref, w1g_ref, b1_ref, w2_ref, b2_ref, erep_ref,
                out_ref, bal_ref, xa_s, pooled_s, gscale_s):
    p = pl.program_id(0)
    b = pl.program_id(1)
    s = pl.program_id(2)
    s_blk = x_ref.shape[1]

    @pl.when(p == 0)
    def _pass1():
        xb = x_ref[0]
        xa_s[b, pl.ds(s * s_blk, s_blk)] = jax.lax.dot_general(
            xb, a_ref[...], (((1,), (1,)), ((), ())),
            preferred_element_type=jnp.float32)
        psum = jnp.sum(xb, axis=0, keepdims=True)

        @pl.when(s == 0)
        def _init():
            pooled_s[pl.ds(b, 1)] = psum

        @pl.when(s != 0)
        def _acc():
            pooled_s[pl.ds(b, 1)] += psum

    @pl.when((p == 1) & (b == 0) & (s == 0))
    def _router():
        pooled = pooled_s[...] * inv_seq
        h = jax.lax.dot_general(pooled, w1x_ref[...], (((1,), (1,)), ((), ())),
                                preferred_element_type=jnp.float32)
        h += jax.lax.dot_general(rel_ref[...], w1r_ref[...],
                                 (((1,), (1,)), ((), ())),
                                 preferred_element_type=jnp.float32)
        h += jax.lax.dot_general(reg_ref[...], w1g_ref[...],
                                 (((1,), (1,)), ((), ())),
                                 preferred_element_type=jnp.float32)
        h += b1_ref[...]
        h = 0.5 * h * (1.0 + jax.lax.erf(h * (1.0 / math.sqrt(2.0))))
        logits = jax.lax.dot_general(h, w2_ref[...], (((1,), (1,)), ((), ())),
                                     preferred_element_type=jnp.float32)
        logits += b2_ref[...]
        m = jnp.max(logits, axis=-1, keepdims=True)
        pr = jnp.exp(logits - m)
        pr = pr / jnp.sum(pr, axis=-1, keepdims=True)
        gscale_s[...] = jax.lax.dot_general(
            pr, erep_ref[...], (((1,), (0,)), ((), ())),
            preferred_element_type=jnp.float32) * SCALING
        avg = jnp.mean(pr, axis=0, keepdims=True)
        bal_ref[...] = BALANCE_COEFF * E * jnp.sum(avg * avg, axis=1,
                                                   keepdims=True)

    @pl.when(p == 1)
    def _pass2():
        xs = xa_s[b, pl.ds(s * s_blk, s_blk)] * gscale_s[pl.ds(b, 1)]
        out_ref[0] = jax.lax.dot_general(
            xs, b_ref[...], (((1,), (0,)), ((), ())),
            preferred_element_type=jnp.float32)


def kernel(x, reliability_vec, regime_vec, lora_A, lora_B, W1, b1, W2, b2):
    B, S, d_in = x.shape
    e, r, _ = lora_A.shape
    er = e * r
    d_out = lora_B.shape[1]
    hid = W1.shape[0]
    ns = S // S_BLK

    a_mat = lora_A.reshape(er, d_in)
    b_mat = lora_B.transpose(0, 2, 1).reshape(er, d_out)
    w1x = W1[:, :d_in]
    w1r = W1[:, d_in:d_in + reliability_vec.shape[1]]
    w1g = W1[:, d_in + reliability_vec.shape[1]:]
    b1_2d = b1.reshape(1, hid)
    b2_2d = b2.reshape(1, e)
    erep = jnp.repeat(jnp.eye(e, dtype=jnp.float32), r, axis=1)

    def x_map(pp, bb, ss):
        return (jnp.where(pp == 0, bb, B - 1),
                jnp.where(pp == 0, ss, ns - 1), 0)

    def out_map(pp, bb, ss):
        return (jnp.where(pp == 1, bb, 0), jnp.where(pp == 1, ss, 0), 0)

    small = [reliability_vec, regime_vec, w1x, w1r, w1g, b1_2d, W2, b2_2d,
             erep]
    out, bal = pl.pallas_call(
        functools.partial(_fused_body, 1.0 / S),
        grid=(2, B, ns),
        in_specs=[pl.BlockSpec((1, S_BLK, d_in), x_map),
                  pl.BlockSpec((er, d_in), lambda pp, bb, ss: (0, 0)),
                  pl.BlockSpec((er, d_out), lambda pp, bb, ss: (0, 0))]
                 + [pl.BlockSpec(a.shape, lambda pp, bb, ss: (0, 0))
                    for a in small],
        out_specs=[
            pl.BlockSpec((1, S_BLK, d_out), out_map),
            pl.BlockSpec((1, 1), lambda pp, bb, ss: (0, 0)),
        ],
        out_shape=[
            jax.ShapeDtypeStruct((B, S, d_out), jnp.float32),
            jax.ShapeDtypeStruct((1, 1), jnp.float32),
        ],
        scratch_shapes=[
            pltpu.VMEM((B, S, er), jnp.float32),
            pltpu.VMEM((B, d_in), jnp.float32),
            pltpu.VMEM((B, er), jnp.float32),
        ],
        compiler_params=pltpu.CompilerParams(
            dimension_semantics=("arbitrary", "parallel", "parallel")),
    )(x, a_mat, b_mat, *small)

    return (out, bal.reshape(()))
